# baseline (speedup 1.0000x reference)
import functools

import jax
import jax.numpy as jnp
from jax import lax
from jax.experimental import pallas as pl
from jax.experimental.pallas import tpu as pltpu
from jax.experimental.pallas import tpu_sc as plsc

DIM = 64
G = 16
NB = 8
ROWS = 8
WROWS = 2

_info = plsc.get_sparse_core_info()
NC, NS = _info.num_cores, _info.num_subcores
NW = NC * NS


def _build(b, s):
    tpw = b * s // NW
    assert tpw % 128 == 0
    trows = tpw // 128
    tpo = ROWS * s
    assert tpo % G == 0 and tpw % tpo == 0 and trows % WROWS == 0
    cpo = tpo // G
    nch = tpw // G
    cpw = WROWS * 128 // G
    mesh = plsc.VectorSubcoreMesh(core_axis_name="c", subcore_axis_name="s")

    @functools.partial(
        pl.kernel,
        mesh=mesh,
        out_type=jax.ShapeDtypeStruct((b, s, DIM), jnp.float32),
        scratch_types=[
            pltpu.VMEM((trows, 128), jnp.int32),
            pltpu.VMEM((trows, 128), jnp.int32),
            pltpu.VMEM((NB, G, 128), jnp.float32),
            pltpu.VMEM((ROWS, s, DIM), jnp.float32),
            pltpu.SemaphoreType.DMA((NB,)),
        ],
        compiler_params=pltpu.CompilerParams(use_tc_tiling_on_sc=True),
    )
    def k(tok_hbm, table_hbm, out_hbm, rrv, tidv, tiles_v, obuf, gsem):
        wid = lax.axis_index("s") * NC + lax.axis_index("c")
        pltpu.sync_copy(tok_hbm.at[wid], rrv)

        def tid_body(kk, carry):
            r = lax.div(kk, 8)
            o = 16 * lax.rem(kk, 8)
            t16 = rrv[r, pl.ds(o, 16)]
            tidv[r, pl.ds(o, 16)] = lax.shift_right_logical(t16, 1)
            return carry

        lax.fori_loop(0, trows * 8, tid_body, 0)

        def idx_slice(c):
            return tidv.at[lax.div(c, 8), pl.ds(16 * lax.rem(c, 8), 16)]

        def issue_gather(c, ring):
            pltpu.async_copy(table_hbm.at[idx_slice(c)], tiles_v.at[ring],
                             gsem.at[ring])

        for p in range(NB):
            issue_gather(p, p)

        def step(c, carry):
            ring = lax.rem(c, NB)
            jj = lax.rem(c, cpo)

            pltpu.make_async_copy(table_hbm.at[idx_slice(c)],
                                  tiles_v.at[ring], gsem.at[ring]).wait()

            col0 = 16 * lax.rem(c, 8)
            t16 = rrv[lax.div(c, 8), pl.ds(col0, 16)]
            hv = lax.shift_left(lax.bitwise_and(t16, 1), 6)
            q0 = jj * G
            for l in range(G):
                rr = hv[l]
                q = q0 + l
                a = lax.div(q, s)
                bb = lax.rem(q, s)
                for v in range(DIM // 16):
                    obuf[a, bb, pl.ds(16 * v, 16)] = (
                        tiles_v[ring, l, pl.ds(rr + 16 * v, 16)])

            @pl.when(c + NB < nch)
            def _():
                issue_gather(c + NB, ring)

            @pl.when(jj == cpo - 1)
            def _():
                m = lax.div(c, cpo)
                pltpu.sync_copy(
                    obuf,
                    out_hbm.at[pl.ds(wid * (tpw // s) + m * ROWS, ROWS)])

            return carry

        lax.fori_loop(0, nch, step, 0)

    return k


def kernel(tokens, table):
    b, s = tokens.shape
    v = table.shape[0]
    tpw = b * s // NW
    tok3 = tokens.reshape(-1).astype(jnp.int32).reshape(NW, tpw // 128, 128)
    table2 = table.reshape(v // 2, 2 * DIM)
    return _build(b, s)(tok3, table2)
